# Initial kernel scaffold; baseline (speedup 1.0000x reference)
import jax
import jax.numpy as jnp
from jax.experimental import pallas as pl


def kernel(input_atom, input_bond, atom_graph, bond_graph, num_nbs, atom_features, W2, b2, W1, b1):
    raise NotImplementedError("write your pallas kernel here")


# speedup vs baseline: 14.5119x; 14.5119x over previous
import functools

import jax
import jax.numpy as jnp
from jax.experimental import pallas as pl

HID = 256
DEPTH = 3
MAX_NB = 10
A = 60
APAD = 64
NB = 600
EB = 5
R = MAX_NB * APAD


def _wl_kernel(af_ref, bondT_ref, ag_ref, bg_ref, nn_ref,
               W2a_ref, W2b_ref, b2_ref, W1_ref, b1_ref, out_ref):
    f32 = jnp.float32
    af = af_ref[0]
    bondT = bondT_ref[0]
    ag = ag_ref[0]
    bg = bg_ref[0]
    nn = nn_ref[0]

    r_col = jax.lax.broadcasted_iota(jnp.int32, (R, 1), 0)
    j_col = r_col // APAD
    mask_col = (j_col < nn).astype(f32)

    t_row = jax.lax.broadcasted_iota(jnp.int32, (R, APAD), 1)
    A1h = jnp.where(ag == t_row, mask_col, 0.0)

    s_col = jax.lax.broadcasted_iota(jnp.int32, (NB, R), 0)
    B1hT = (s_col == bg).astype(f32)
    fbT = jnp.dot(bondT, B1hT, preferred_element_type=f32)
    fb = fbT.T
    bond_c = (jnp.dot(fb, W2b_ref[...], preferred_element_type=f32)
              + b2_ref[...]) * mask_col

    a_col = jax.lax.broadcasted_iota(jnp.int32, (APAD, 1), 0)
    row_valid = (a_col < A).astype(f32)

    W2a = W2a_ref[...]
    W1 = W1_ref[...]
    b1 = b1_ref[...]
    for _ in range(DEPTH):
        afW = jnp.dot(af, W2a, preferred_element_type=f32)
        g = jnp.dot(A1h, afW, preferred_element_type=f32)
        pre = jax.nn.relu(g + bond_c)
        nei = pre[0:APAD]
        for j in range(1, MAX_NB):
            nei = nei + pre[j * APAD:(j + 1) * APAD]
        nl = jnp.concatenate([af, nei], axis=1)
        af = jax.nn.relu(jnp.dot(nl, W1, preferred_element_type=f32) + b1)
        af = af * row_valid
    out_ref[0, 0, :] = jnp.sum(af, axis=0)


@jax.jit
def kernel(input_atom, input_bond, atom_graph, bond_graph, num_nbs,
           atom_features, W2, b2, W1, b1):
    del input_atom
    B = atom_features.shape[0]

    ag = atom_graph[..., 0].astype(jnp.int32)
    ag_p = jnp.transpose(ag, (0, 2, 1))
    ag_p = jnp.pad(ag_p, ((0, 0), (0, 0), (0, APAD - A)))
    ag_p = ag_p.reshape(B, R, 1)
    bg = bond_graph[..., 0].astype(jnp.int32)
    bg_p = jnp.transpose(bg, (0, 2, 1))
    bg_p = jnp.pad(bg_p, ((0, 0), (0, 0), (0, APAD - A)))
    bg_p = bg_p.reshape(B, 1, R)
    nn_p = jnp.pad(num_nbs.astype(jnp.int32), ((0, 0), (0, APAD - A)))
    nn_p = jnp.tile(nn_p, (1, MAX_NB)).reshape(B, R, 1)

    af0 = jnp.pad(atom_features, ((0, 0), (0, APAD - A), (0, 0)))
    bondT = jnp.transpose(input_bond, (0, 2, 1))

    W2a = W2[:HID]
    W2b = W2[HID:]
    b2r = b2.reshape(1, HID)
    b1r = b1.reshape(1, HID)

    rep2 = lambda i: (0, 0)
    out = pl.pallas_call(
        _wl_kernel,
        grid=(B,),
        in_specs=[
            pl.BlockSpec((1, APAD, HID), lambda i: (i, 0, 0)),
            pl.BlockSpec((1, EB, NB), lambda i: (i, 0, 0)),
            pl.BlockSpec((1, R, 1), lambda i: (i, 0, 0)),
            pl.BlockSpec((1, 1, R), lambda i: (i, 0, 0)),
            pl.BlockSpec((1, R, 1), lambda i: (i, 0, 0)),
            pl.BlockSpec((HID, HID), rep2),
            pl.BlockSpec((EB, HID), rep2),
            pl.BlockSpec((1, HID), rep2),
            pl.BlockSpec((2 * HID, HID), rep2),
            pl.BlockSpec((1, HID), rep2),
        ],
        out_specs=pl.BlockSpec((1, 1, HID), lambda i: (i, 0, 0)),
        out_shape=jax.ShapeDtypeStruct((B, 1, HID), jnp.float32),
    )(af0, bondT, ag_p, bg_p, nn_p, W2a, W2b, b2r, W1, b1r)
    return out.reshape(B, HID)


# speedup vs baseline: 24.1008x; 1.6608x over previous
import jax
import jax.numpy as jnp
from jax.experimental import pallas as pl

HID = 256
DEPTH = 3
MAX_NB = 10
A = 60
APAD = 64
NB = 600
EB = 5
R = MAX_NB * APAD
P = 4


def _wl_kernel(af_ref, bondT_ref, ag_ref, bg_ref, nn_ref,
               W2a_ref, W2b_ref, b2_ref, W1_ref, b1_ref, out_ref):
    f32 = jnp.float32
    af = af_ref[...].reshape(P * APAD, HID)
    ag = ag_ref[...].reshape(P * R, 1)
    nn = nn_ref[...].reshape(P * R, 1)

    r_col = jax.lax.broadcasted_iota(jnp.int32, (P * R, 1), 0)
    j_col = (r_col % R) // APAD
    mask_col = (j_col < nn).astype(f32)

    t_row = jax.lax.broadcasted_iota(jnp.int32, (R, APAD), 1)
    A1h = [jnp.where(ag[m * R:(m + 1) * R] == t_row,
                     mask_col[m * R:(m + 1) * R], 0.0)
           for m in range(P)]

    s_col = jax.lax.broadcasted_iota(jnp.int32, (NB, R), 0)
    W2b = W2b_ref[...]
    b2 = b2_ref[...]
    bond_parts = []
    for m in range(P):
        bg_m = bg_ref[m]
        B1hT = (s_col == bg_m).astype(f32)
        fbT = jnp.dot(bondT_ref[m], B1hT, preferred_element_type=f32)
        fb = fbT.T
        bond_parts.append(jnp.dot(fb, W2b, preferred_element_type=f32))
    bond_c = (jnp.concatenate(bond_parts, axis=0) + b2) * mask_col

    a_col = jax.lax.broadcasted_iota(jnp.int32, (P * APAD, 1), 0)
    row_valid = ((a_col % APAD) < A).astype(f32)

    W2a = W2a_ref[...]
    W1 = W1_ref[...]
    b1 = b1_ref[...]
    for _ in range(DEPTH):
        afW = jnp.dot(af, W2a, preferred_element_type=f32)
        g = jnp.concatenate(
            [jnp.dot(A1h[m], afW[m * APAD:(m + 1) * APAD],
                     preferred_element_type=f32)
             for m in range(P)], axis=0)
        pre = jax.nn.relu(g + bond_c)
        nei_parts = []
        for m in range(P):
            nei = pre[m * R:m * R + APAD]
            for j in range(1, MAX_NB):
                base = m * R + j * APAD
                nei = nei + pre[base:base + APAD]
            nei_parts.append(nei)
        nei = jnp.concatenate(nei_parts, axis=0)
        nl = jnp.concatenate([af, nei], axis=1)
        af = jax.nn.relu(jnp.dot(nl, W1, preferred_element_type=f32) + b1)
        af = af * row_valid
    for m in range(P):
        out_ref[m, 0, :] = jnp.sum(af[m * APAD:(m + 1) * APAD], axis=0)


@jax.jit
def kernel(input_atom, input_bond, atom_graph, bond_graph, num_nbs,
           atom_features, W2, b2, W1, b1):
    del input_atom
    B = atom_features.shape[0]

    ag = atom_graph[..., 0].astype(jnp.int32)
    ag_p = jnp.transpose(ag, (0, 2, 1))
    ag_p = jnp.pad(ag_p, ((0, 0), (0, 0), (0, APAD - A)))
    ag_p = ag_p.reshape(B, R, 1)
    bg = bond_graph[..., 0].astype(jnp.int32)
    bg_p = jnp.transpose(bg, (0, 2, 1))
    bg_p = jnp.pad(bg_p, ((0, 0), (0, 0), (0, APAD - A)))
    bg_p = bg_p.reshape(B, 1, R)
    nn_p = jnp.pad(num_nbs.astype(jnp.int32), ((0, 0), (0, APAD - A)))
    nn_p = jnp.tile(nn_p, (1, MAX_NB)).reshape(B, R, 1)

    af0 = jnp.pad(atom_features, ((0, 0), (0, APAD - A), (0, 0)))
    bondT = jnp.transpose(input_bond, (0, 2, 1))

    W2a = W2[:HID]
    W2b = W2[HID:]
    b2r = b2.reshape(1, HID)
    b1r = b1.reshape(1, HID)

    rep2 = lambda i: (0, 0)
    out = pl.pallas_call(
        _wl_kernel,
        grid=(B // P,),
        in_specs=[
            pl.BlockSpec((P, APAD, HID), lambda i: (i, 0, 0)),
            pl.BlockSpec((P, EB, NB), lambda i: (i, 0, 0)),
            pl.BlockSpec((P, R, 1), lambda i: (i, 0, 0)),
            pl.BlockSpec((P, 1, R), lambda i: (i, 0, 0)),
            pl.BlockSpec((P, R, 1), lambda i: (i, 0, 0)),
            pl.BlockSpec((HID, HID), rep2),
            pl.BlockSpec((EB, HID), rep2),
            pl.BlockSpec((1, HID), rep2),
            pl.BlockSpec((2 * HID, HID), rep2),
            pl.BlockSpec((1, HID), rep2),
        ],
        out_specs=pl.BlockSpec((P, 1, HID), lambda i: (i, 0, 0)),
        out_shape=jax.ShapeDtypeStruct((B, 1, HID), jnp.float32),
    )(af0, bondT, ag_p, bg_p, nn_p, W2a, W2b, b2r, W1, b1r)
    return out.reshape(B, HID)


# speedup vs baseline: 25.5179x; 1.0588x over previous
import jax
import jax.numpy as jnp
from jax.experimental import pallas as pl

HID = 256
DEPTH = 3
MAX_NB = 10
A = 60
APAD = 64
NB = 600
EB = 5
R = MAX_NB * APAD
P = 4


def _wl_kernel(af_ref, bondT_ref, ag_ref, bg_ref, nn_ref,
               W2a_ref, W2b_ref, b2_ref, W1_ref, b1_ref, out_ref):
    f32 = jnp.float32
    af = af_ref[...].reshape(P * APAD, HID)
    ag = ag_ref[...].reshape(P * R, 1)
    nn = nn_ref[...].reshape(P * R, 1)

    r_col = jax.lax.broadcasted_iota(jnp.int32, (P * R, 1), 0)
    j_col = (r_col % R) // APAD
    mask_col = (j_col < nn).astype(f32)

    t_row = jax.lax.broadcasted_iota(jnp.int32, (R, APAD), 1)
    s_col = jax.lax.broadcasted_iota(jnp.int32, (NB, R), 0)
    A1hx = []
    for m in range(P):
        mask_m = mask_col[m * R:(m + 1) * R]
        A1h = jnp.where(ag[m * R:(m + 1) * R] == t_row, mask_m, 0.0)
        B1hT = (s_col == bg_ref[m]).astype(f32)
        fbT = jnp.dot(bondT_ref[m], B1hT, preferred_element_type=f32)
        fb = fbT.T * mask_m
        A1hx.append(jnp.concatenate([A1h, fb, mask_m], axis=1))

    W2b_b2 = jnp.concatenate([W2b_ref[...], b2_ref[...]], axis=0)

    a_col = jax.lax.broadcasted_iota(jnp.int32, (P * APAD, 1), 0)
    row_valid = ((a_col % APAD) < A).astype(f32)

    W2a = W2a_ref[...]
    W1 = W1_ref[...]
    b1 = b1_ref[...]
    for _ in range(DEPTH):
        afW = jnp.dot(af, W2a, preferred_element_type=f32)
        pre = jax.nn.relu(jnp.concatenate(
            [jnp.dot(A1hx[m],
                     jnp.concatenate(
                         [afW[m * APAD:(m + 1) * APAD], W2b_b2], axis=0),
                     preferred_element_type=f32)
             for m in range(P)], axis=0))
        nei_parts = []
        for m in range(P):
            nei = pre[m * R:m * R + APAD]
            for j in range(1, MAX_NB):
                base = m * R + j * APAD
                nei = nei + pre[base:base + APAD]
            nei_parts.append(nei)
        nei = jnp.concatenate(nei_parts, axis=0)
        nl = jnp.concatenate([af, nei], axis=1)
        af = jax.nn.relu(jnp.dot(nl, W1, preferred_element_type=f32) + b1)
        af = af * row_valid
    for m in range(P):
        out_ref[m, 0, :] = jnp.sum(af[m * APAD:(m + 1) * APAD], axis=0)


@jax.jit
def kernel(input_atom, input_bond, atom_graph, bond_graph, num_nbs,
           atom_features, W2, b2, W1, b1):
    del input_atom
    B = atom_features.shape[0]

    ag = atom_graph[..., 0].astype(jnp.int32)
    ag_p = jnp.transpose(ag, (0, 2, 1))
    ag_p = jnp.pad(ag_p, ((0, 0), (0, 0), (0, APAD - A)))
    ag_p = ag_p.reshape(B, R, 1)
    bg = bond_graph[..., 0].astype(jnp.int32)
    bg_p = jnp.transpose(bg, (0, 2, 1))
    bg_p = jnp.pad(bg_p, ((0, 0), (0, 0), (0, APAD - A)))
    bg_p = bg_p.reshape(B, 1, R)
    nn_p = jnp.pad(num_nbs.astype(jnp.int32), ((0, 0), (0, APAD - A)))
    nn_p = jnp.tile(nn_p, (1, MAX_NB)).reshape(B, R, 1)

    af0 = jnp.pad(atom_features, ((0, 0), (0, APAD - A), (0, 0)))
    bondT = jnp.transpose(input_bond, (0, 2, 1))

    W2a = W2[:HID]
    W2b = W2[HID:]
    b2r = b2.reshape(1, HID)
    b1r = b1.reshape(1, HID)

    rep2 = lambda i: (0, 0)
    out = pl.pallas_call(
        _wl_kernel,
        grid=(B // P,),
        in_specs=[
            pl.BlockSpec((P, APAD, HID), lambda i: (i, 0, 0)),
            pl.BlockSpec((P, EB, NB), lambda i: (i, 0, 0)),
            pl.BlockSpec((P, R, 1), lambda i: (i, 0, 0)),
            pl.BlockSpec((P, 1, R), lambda i: (i, 0, 0)),
            pl.BlockSpec((P, R, 1), lambda i: (i, 0, 0)),
            pl.BlockSpec((HID, HID), rep2),
            pl.BlockSpec((EB, HID), rep2),
            pl.BlockSpec((1, HID), rep2),
            pl.BlockSpec((2 * HID, HID), rep2),
            pl.BlockSpec((1, HID), rep2),
        ],
        out_specs=pl.BlockSpec((P, 1, HID), lambda i: (i, 0, 0)),
        out_shape=jax.ShapeDtypeStruct((B, 1, HID), jnp.float32),
    )(af0, bondT, ag_p, bg_p, nn_p, W2a, W2b, b2r, W1, b1r)
    return out.reshape(B, HID)
